# baseline (speedup 1.0000x reference)
import functools

import jax
import jax.numpy as jnp
from jax import lax
from jax.experimental import pallas as pl
from jax.experimental.pallas import tpu as pltpu
from jax.experimental.pallas import tpu_sc as plsc

D = 16


def _make_sc_kernel(batch: int):
    info = plsc.get_sparse_core_info()
    nc, ns, nl = info.num_cores, info.num_subcores, info.num_lanes
    nw = nc * ns
    assert nl == D
    assert batch % (8 * nw) == 0
    chunk = batch // nw

    mesh = plsc.VectorSubcoreMesh(core_axis_name="c", subcore_axis_name="s")

    @functools.partial(
        pl.kernel,
        mesh=mesh,
        out_type=jax.ShapeDtypeStruct((batch,), jnp.float32),
        scratch_types=[
            pltpu.VMEM((chunk,), jnp.int32),
            pltpu.VMEM((chunk,), jnp.int32),
            pltpu.VMEM((chunk, D), jnp.float32),
            pltpu.VMEM((chunk, D), jnp.float32),
            pltpu.VMEM((chunk,), jnp.float32),
            pltpu.SemaphoreType.DMA,
            pltpu.SemaphoreType.DMA,
        ],
        compiler_params=pltpu.CompilerParams(
            needs_layout_passes=False, use_tc_tiling_on_sc=False),
    )
    def sc_kernel(uid_idx_hbm, iid_idx_hbm, uid_tab_hbm, iid_tab_hbm,
                  out_hbm, idx_u, idx_i, rows_u, rows_i, out_v, sem_u, sem_i):
        wid = lax.axis_index("s") * nc + lax.axis_index("c")
        base = wid * chunk

        pltpu.sync_copy(uid_idx_hbm.at[pl.ds(base, chunk)], idx_u)
        pltpu.sync_copy(iid_idx_hbm.at[pl.ds(base, chunk)], idx_i)
        cu = pltpu.async_copy(uid_tab_hbm.at[idx_u], rows_u, sem_u)
        ci = pltpu.async_copy(iid_tab_hbm.at[idx_i], rows_i, sem_i)
        cu.wait()
        ci.wait()

        lane = lax.iota(jnp.int32, D)

        def body(g, carry):
            acc = jnp.zeros((D,), jnp.float32)
            for k in range(D):
                i = g * D + k
                u = rows_u[i, :]
                v = rows_i[i, :]
                s = jnp.sum(u * v)
                acc = jnp.where(lane == k, s, acc)
            out_v[pl.ds(g * D, D)] = acc
            return carry

        lax.fori_loop(0, chunk // D, body, 0)

        pltpu.sync_copy(out_v, out_hbm.at[pl.ds(base, chunk)])

    return sc_kernel


def kernel(x, uid_table, iid_table):
    batch = x.shape[0]
    xi = x.astype(jnp.int32)
    uid_idx = xi[:, 0]
    iid_idx = xi[:, 1]
    sc = _make_sc_kernel(batch)
    return sc(uid_idx, iid_idx, uid_table, iid_table)
